# baseline (speedup 1.0000x reference)
import functools

import jax
import jax.numpy as jnp
import numpy as np
from jax import lax
from jax.experimental import pallas as pl
from jax.experimental.pallas import tpu as pltpu
from jax.experimental.pallas import tpu_sc as plsc

M = 4096
N = 32768
L = 16
D = 12
CAP = 2048
IMAX = np.int32(2**31 - 1)


def _searchsorted(ot_v, target):
    pos = jnp.zeros((L,), jnp.int32)
    bit = N
    while bit >= 1:
        nxt = pos + bit
        ok = nxt <= N
        idx = jnp.minimum(nxt, N) - 1
        vals = plsc.load_gather(ot_v, [idx])
        pos = jnp.where(ok & (vals < target), nxt, pos)
        bit //= 2
    return pos


def _body(nc, qpw, vt_hbm, ot_hbm, odomt_hbm, out_hbm, *refs):
    cols_sh = refs[:D]
    (ot_v, vt_v, a_v, b_v, w0_v, w1_v, y0t_v, y1t_v, win_v, out_v,
     sem0, sem1) = refs[D:]
    sid = lax.axis_index("s")
    wid = sid * nc + lax.axis_index("c")
    base = wid * qpw

    with jax.named_scope("stage_odom"):
        for g in range(D):
            @pl.when(sid == g)
            def _(g=g):
                pltpu.sync_copy(odomt_hbm.at[g], cols_sh[g])

    with jax.named_scope("stage_table"):
        pltpu.sync_copy(ot_hbm, ot_v)
        pltpu.sync_copy(vt_hbm.at[pl.ds(base, qpw)], vt_v)

    _scope = jax.named_scope("search")
    _scope.__enter__()
    rmin = jnp.full((L,), IMAX, jnp.int32)
    rmax = jnp.zeros((L,), jnp.int32)
    for k in range(qpw // L):
        vt16 = vt_v[pl.ds(k * L, L)]
        pL = _searchsorted(ot_v, vt16)
        wL = plsc.load_gather(ot_v, [jnp.maximum(pL, 1) - 1])
        wR = plsc.load_gather(ot_v, [jnp.minimum(pL, N - 1)])
        dL = vt16 - wL
        dR = jnp.where(pL < N, wR - vt16, IMAX)
        takeL = dL <= dR
        first_wL = _searchsorted(ot_v, wL)
        ref = jnp.where(takeL, first_wL, pL)
        d = jnp.where(takeL, dL, -dR)
        step = (d > 0).astype(jnp.int32) - (d < 0).astype(jnp.int32)
        q = jnp.clip(ref + step, 0, M - 1)
        a = jnp.minimum(ref, q)
        b = jnp.maximum(ref, q)
        x0 = plsc.load_gather(ot_v, [a])
        x1 = plsc.load_gather(ot_v, [b])
        eq = x0 == x1
        x0f = x0.astype(jnp.float32)
        x1f = x1.astype(jnp.float32)
        vtf = vt16.astype(jnp.float32)
        denom = jnp.where(eq, jnp.float32(1.0), x1f - x0f)
        w0 = 1.0 - (vtf - x0f) / denom
        w1 = 1.0 - w0
        w0 = jnp.where(eq, jnp.float32(1.0), w0)
        w1 = jnp.where(eq, jnp.float32(0.0), w1)
        a_v[pl.ds(k * L, L)] = a
        b_v[pl.ds(k * L, L)] = b
        w0_v[pl.ds(k * L, L)] = w0
        w1_v[pl.ds(k * L, L)] = w1
        rmin = jnp.minimum(rmin, a)
        rmax = jnp.maximum(rmax, b)
    lo = jnp.min(rmin)
    hi = jnp.max(rmax)
    lo2 = pl.multiple_of(jnp.minimum(lo & ~7, N - CAP), 8)
    fits = hi < lo2 + CAP
    _scope.__exit__(None, None, None)

    with jax.named_scope("barrier"):
        plsc.subcore_barrier()

    with jax.named_scope("gather_rows"):
        @pl.when(fits)
        def _():
            pltpu.sync_copy(odomt_hbm.at[:, pl.ds(lo2, CAP)], win_v)
            lane0 = lax.iota(jnp.int32, L)
            for k in range(qpw // L):
                a16 = a_v[pl.ds(k * L, L)] - lo2
                b16 = b_v[pl.ds(k * L, L)] - lo2
                for g in range(D):
                    g16 = jnp.full((L,), g, jnp.int32)
                    y0t_v[g, pl.ds(k * L, L)] = plsc.load_gather(
                        win_v, [g16, a16])
                    y1t_v[g, pl.ds(k * L, L)] = plsc.load_gather(
                        win_v, [g16, b16])

        @pl.when(jnp.logical_not(fits))
        def _():
            handles = []
            for g in range(D):
                handles.append(
                    pltpu.async_copy(cols_sh[g].at[a_v], y0t_v.at[g], sem0))
                handles.append(
                    pltpu.async_copy(cols_sh[g].at[b_v], y1t_v.at[g], sem1))
            for h in handles:
                h.wait()

    with jax.named_scope("lerp"):
        lane = lax.iota(jnp.int32, L)
        for k in range(qpw // L):
            s0 = w0_v[pl.ds(k * L, L)]
            s1 = w1_v[pl.ds(k * L, L)]
            row = lane + (k * L)
            for g in range(D):
                y0 = y0t_v[g, pl.ds(k * L, L)]
                y1 = y1t_v[g, pl.ds(k * L, L)]
                col = jnp.full((L,), g, jnp.int32)
                plsc.store_scatter(out_v, [row, col], y0 * s0 + y1 * s1)

    with jax.named_scope("writeback"):
        pltpu.sync_copy(out_v, out_hbm.at[pl.ds(base, qpw)])


@jax.jit
def _run(vt, ot, odomt):
    info = plsc.get_sparse_core_info()
    nc, ns = info.num_cores, info.num_subcores
    nw = nc * ns
    qpw = M // nw
    mesh = plsc.VectorSubcoreMesh(core_axis_name="c", subcore_axis_name="s")
    run = pl.kernel(
        functools.partial(_body, nc, qpw),
        out_type=jax.ShapeDtypeStruct((M, D), jnp.float32),
        mesh=mesh,
        compiler_params=pltpu.CompilerParams(
            needs_layout_passes=False, use_tc_tiling_on_sc=False),
        scratch_types=[pltpu.VMEM_SHARED((N,), jnp.float32)] * D + [
            pltpu.VMEM((N,), jnp.int32),
            pltpu.VMEM((qpw,), jnp.int32),
            pltpu.VMEM((qpw,), jnp.int32),
            pltpu.VMEM((qpw,), jnp.int32),
            pltpu.VMEM((qpw,), jnp.float32),
            pltpu.VMEM((qpw,), jnp.float32),
            pltpu.VMEM((D, qpw), jnp.float32),
            pltpu.VMEM((D, qpw), jnp.float32),
            pltpu.VMEM((D, CAP), jnp.float32),
            pltpu.VMEM((qpw, D), jnp.float32),
            pltpu.SemaphoreType.DMA,
            pltpu.SemaphoreType.DMA,
        ],
    )
    return run(vt, ot, odomt)


def kernel(valid_timestamps, odom_timestamps, odom):
    return _run(valid_timestamps, odom_timestamps, odom.T)


# speedup vs baseline: 1.1609x; 1.1609x over previous
import functools

import jax
import jax.numpy as jnp
import numpy as np
from jax import lax
from jax.experimental import pallas as pl
from jax.experimental.pallas import tpu as pltpu
from jax.experimental.pallas import tpu_sc as plsc

M = 4096
N = 32768
L = 16
D = 12
CAP = 2048
SPEC = (M - 1) & ~7
IMAX = np.int32(2**31 - 1)


def _searchsorted(ot_v, target):
    pos = jnp.zeros((L,), jnp.int32)
    bit = N
    while bit >= 1:
        nxt = pos + bit
        ok = nxt <= N
        idx = jnp.minimum(nxt, N) - 1
        vals = plsc.load_gather(ot_v, [idx])
        pos = jnp.where(ok & (vals < target), nxt, pos)
        bit //= 2
    return pos


def _body(nc, qpw, vt_hbm, ot_hbm, odomt_hbm, out_hbm, *refs):
    cols_sh = refs[:D]
    (ot_v, vt_v, a_v, b_v, w0_v, w1_v, y0t_v, y1t_v, win_v, spec_v, out_v,
     sem0, sem1) = refs[D:]
    sid = lax.axis_index("s")
    wid = sid * nc + lax.axis_index("c")
    base = wid * qpw

    with jax.named_scope("stage_odom"):
        for g in range(D):
            @pl.when(sid == g)
            def _(g=g):
                pltpu.sync_copy(odomt_hbm.at[g], cols_sh[g])

    with jax.named_scope("stage_table"):
        pltpu.sync_copy(ot_hbm, ot_v)
        pltpu.sync_copy(vt_hbm.at[pl.ds(base, qpw)], vt_v)

    _scope = jax.named_scope("search")
    _scope.__enter__()
    rmin = jnp.full((L,), IMAX, jnp.int32)
    rmax = jnp.zeros((L,), jnp.int32)
    for k in range(qpw // L):
        vt16 = vt_v[pl.ds(k * L, L)]
        pL = _searchsorted(ot_v, vt16)
        wL = plsc.load_gather(ot_v, [jnp.maximum(pL, 1) - 1])
        wR = plsc.load_gather(ot_v, [jnp.minimum(pL, N - 1)])
        dL = vt16 - wL
        dR = jnp.where(pL < N, wR - vt16, IMAX)
        takeL = dL <= dR
        first_wL = _searchsorted(ot_v, wL)
        ref = jnp.where(takeL, first_wL, pL)
        d = jnp.where(takeL, dL, -dR)
        step = (d > 0).astype(jnp.int32) - (d < 0).astype(jnp.int32)
        q = jnp.clip(ref + step, 0, M - 1)
        a = jnp.minimum(ref, q)
        b = jnp.maximum(ref, q)
        x0 = plsc.load_gather(ot_v, [a])
        x1 = plsc.load_gather(ot_v, [b])
        eq = x0 == x1
        x0f = x0.astype(jnp.float32)
        x1f = x1.astype(jnp.float32)
        vtf = vt16.astype(jnp.float32)
        denom = jnp.where(eq, jnp.float32(1.0), x1f - x0f)
        w0 = 1.0 - (vtf - x0f) / denom
        w1 = 1.0 - w0
        w0 = jnp.where(eq, jnp.float32(1.0), w0)
        w1 = jnp.where(eq, jnp.float32(0.0), w1)
        a_v[pl.ds(k * L, L)] = a
        b_v[pl.ds(k * L, L)] = b
        w0_v[pl.ds(k * L, L)] = w0
        w1_v[pl.ds(k * L, L)] = w1
        rmin = jnp.minimum(rmin, jnp.where(ref >= M, b, a))
        rmax = jnp.maximum(rmax, b)
    lo = jnp.min(rmin)
    hi = jnp.max(rmax)
    lo2 = pl.multiple_of(jnp.minimum(lo & ~7, N - CAP), 8)
    fits = hi < lo2 + CAP
    _scope.__exit__(None, None, None)

    with jax.named_scope("barrier"):
        plsc.subcore_barrier()

    with jax.named_scope("gather_rows"):
        @pl.when(fits)
        def _():
            pltpu.sync_copy(odomt_hbm.at[:, pl.ds(lo2, CAP)], win_v)
            pltpu.sync_copy(odomt_hbm.at[:, pl.ds(SPEC, L)], spec_v)
            for k in range(qpw // L):
                a16 = a_v[pl.ds(k * L, L)]
                b16 = b_v[pl.ds(k * L, L)] - lo2
                isclip = a16 == M - 1
                a16c = jnp.clip(a16 - lo2, 0, CAP - 1)
                for g in range(D):
                    g16 = jnp.full((L,), g, jnp.int32)
                    y0 = plsc.load_gather(win_v, [g16, a16c])
                    sp = spec_v[g][M - 1 - SPEC]
                    y0t_v[g, pl.ds(k * L, L)] = jnp.where(isclip, sp, y0)
                    y1t_v[g, pl.ds(k * L, L)] = plsc.load_gather(
                        win_v, [g16, b16])

        @pl.when(jnp.logical_not(fits))
        def _():
            handles = []
            for g in range(D):
                handles.append(
                    pltpu.async_copy(cols_sh[g].at[a_v], y0t_v.at[g], sem0))
                handles.append(
                    pltpu.async_copy(cols_sh[g].at[b_v], y1t_v.at[g], sem1))
            for h in handles:
                h.wait()

    with jax.named_scope("lerp"):
        lane = lax.iota(jnp.int32, L)
        for k in range(qpw // L):
            s0 = w0_v[pl.ds(k * L, L)]
            s1 = w1_v[pl.ds(k * L, L)]
            row = lane + (k * L)
            for g in range(D):
                y0 = y0t_v[g, pl.ds(k * L, L)]
                y1 = y1t_v[g, pl.ds(k * L, L)]
                col = jnp.full((L,), g, jnp.int32)
                plsc.store_scatter(out_v, [row, col], y0 * s0 + y1 * s1)

    with jax.named_scope("writeback"):
        pltpu.sync_copy(out_v, out_hbm.at[pl.ds(base, qpw)])


@jax.jit
def _run(vt, ot, odomt):
    info = plsc.get_sparse_core_info()
    nc, ns = info.num_cores, info.num_subcores
    nw = nc * ns
    qpw = M // nw
    mesh = plsc.VectorSubcoreMesh(core_axis_name="c", subcore_axis_name="s")
    run = pl.kernel(
        functools.partial(_body, nc, qpw),
        out_type=jax.ShapeDtypeStruct((M, D), jnp.float32),
        mesh=mesh,
        compiler_params=pltpu.CompilerParams(
            needs_layout_passes=False, use_tc_tiling_on_sc=False),
        scratch_types=[pltpu.VMEM_SHARED((N,), jnp.float32)] * D + [
            pltpu.VMEM((N,), jnp.int32),
            pltpu.VMEM((qpw,), jnp.int32),
            pltpu.VMEM((qpw,), jnp.int32),
            pltpu.VMEM((qpw,), jnp.int32),
            pltpu.VMEM((qpw,), jnp.float32),
            pltpu.VMEM((qpw,), jnp.float32),
            pltpu.VMEM((D, qpw), jnp.float32),
            pltpu.VMEM((D, qpw), jnp.float32),
            pltpu.VMEM((D, CAP), jnp.float32),
            pltpu.VMEM((D, L), jnp.float32),
            pltpu.VMEM((qpw, D), jnp.float32),
            pltpu.SemaphoreType.DMA,
            pltpu.SemaphoreType.DMA,
        ],
    )
    return run(vt, ot, odomt)


def kernel(valid_timestamps, odom_timestamps, odom):
    return _run(valid_timestamps, odom_timestamps, odom.T)


# speedup vs baseline: 1.2081x; 1.0407x over previous
import functools

import jax
import jax.numpy as jnp
import numpy as np
from jax import lax
from jax.experimental import pallas as pl
from jax.experimental.pallas import tpu as pltpu
from jax.experimental.pallas import tpu_sc as plsc

M = 4096
N = 32768
L = 16
D = 12
CAP = 2048
SPEC = (M - 1) & ~7
IMAX = np.int32(2**31 - 1)


def _searchsorted(ot_v, target):
    pos = jnp.zeros((L,), jnp.int32)
    bit = N
    while bit >= 1:
        nxt = pos + bit
        ok = nxt <= N
        idx = jnp.minimum(nxt, N) - 1
        vals = plsc.load_gather(ot_v, [idx])
        pos = jnp.where(ok & (vals < target), nxt, pos)
        bit //= 2
    return pos


def _body(nc, qpw, vt_hbm, ot_hbm, odomt_hbm, out_hbm, *refs):
    cols_sh = refs[:D]
    (ot_v, vt_v, a_v, b_v, w0_v, w1_v, y0t_v, y1t_v, win_v, spec_v, out_v,
     sem0, sem1, sem2) = refs[D:]
    sid = lax.axis_index("s")
    wid = sid * nc + lax.axis_index("c")
    base = wid * qpw

    with jax.named_scope("stage_odom"):
        for g in range(D):
            @pl.when(sid == g)
            def _(g=g):
                pltpu.async_copy(odomt_hbm.at[g], cols_sh[g], sem2)

    with jax.named_scope("stage_table"):
        pltpu.sync_copy(ot_hbm, ot_v)
        pltpu.sync_copy(vt_hbm.at[pl.ds(base, qpw)], vt_v)

    _scope = jax.named_scope("search")
    _scope.__enter__()
    rmin = jnp.full((L,), IMAX, jnp.int32)
    rmax = jnp.zeros((L,), jnp.int32)
    for k in range(qpw // L):
        vt16 = vt_v[pl.ds(k * L, L)]
        pL = _searchsorted(ot_v, vt16)
        wL = plsc.load_gather(ot_v, [jnp.maximum(pL, 1) - 1])
        wR = plsc.load_gather(ot_v, [jnp.minimum(pL, N - 1)])
        dL = vt16 - wL
        dR = jnp.where(pL < N, wR - vt16, IMAX)
        takeL = dL <= dR
        first_wL = _searchsorted(ot_v, wL)
        ref = jnp.where(takeL, first_wL, pL)
        d = jnp.where(takeL, dL, -dR)
        step = (d > 0).astype(jnp.int32) - (d < 0).astype(jnp.int32)
        q = jnp.clip(ref + step, 0, M - 1)
        a = jnp.minimum(ref, q)
        b = jnp.maximum(ref, q)
        x0 = plsc.load_gather(ot_v, [a])
        x1 = plsc.load_gather(ot_v, [b])
        eq = x0 == x1
        x0f = x0.astype(jnp.float32)
        x1f = x1.astype(jnp.float32)
        vtf = vt16.astype(jnp.float32)
        denom = jnp.where(eq, jnp.float32(1.0), x1f - x0f)
        w0 = 1.0 - (vtf - x0f) / denom
        w1 = 1.0 - w0
        w0 = jnp.where(eq, jnp.float32(1.0), w0)
        w1 = jnp.where(eq, jnp.float32(0.0), w1)
        a_v[pl.ds(k * L, L)] = a
        b_v[pl.ds(k * L, L)] = b
        w0_v[pl.ds(k * L, L)] = w0
        w1_v[pl.ds(k * L, L)] = w1
        rmin = jnp.minimum(rmin, jnp.where(ref >= M, b, a))
        rmax = jnp.maximum(rmax, b)
    lo = jnp.min(rmin)
    hi = jnp.max(rmax)
    lo2 = pl.multiple_of(jnp.minimum(lo & ~7, N - CAP), 8)
    fits = hi < lo2 + CAP
    _scope.__exit__(None, None, None)

    with jax.named_scope("barrier"):
        for g in range(D):
            @pl.when(sid == g)
            def _(g=g):
                pltpu.make_async_copy(odomt_hbm.at[g], cols_sh[g], sem2).wait()
        plsc.subcore_barrier()

    with jax.named_scope("gather_rows"):
        @pl.when(fits)
        def _():
            pltpu.sync_copy(odomt_hbm.at[:, pl.ds(lo2, CAP)], win_v)
            pltpu.sync_copy(odomt_hbm.at[:, pl.ds(SPEC, L)], spec_v)
            for k in range(qpw // L):
                a16 = a_v[pl.ds(k * L, L)]
                b16 = b_v[pl.ds(k * L, L)] - lo2
                isclip = a16 == M - 1
                a16c = jnp.clip(a16 - lo2, 0, CAP - 1)
                for g in range(D):
                    g16 = jnp.full((L,), g, jnp.int32)
                    y0 = plsc.load_gather(win_v, [g16, a16c])
                    sp = spec_v[g][M - 1 - SPEC]
                    y0t_v[g, pl.ds(k * L, L)] = jnp.where(isclip, sp, y0)
                    y1t_v[g, pl.ds(k * L, L)] = plsc.load_gather(
                        win_v, [g16, b16])

        @pl.when(jnp.logical_not(fits))
        def _():
            handles = []
            for g in range(D):
                handles.append(
                    pltpu.async_copy(cols_sh[g].at[a_v], y0t_v.at[g], sem0))
                handles.append(
                    pltpu.async_copy(cols_sh[g].at[b_v], y1t_v.at[g], sem1))
            for h in handles:
                h.wait()

    with jax.named_scope("lerp"):
        for k in range(qpw // L):
            s0 = w0_v[pl.ds(k * L, L)]
            s1 = w1_v[pl.ds(k * L, L)]
            for g in range(D):
                y0 = y0t_v[g, pl.ds(k * L, L)]
                y1 = y1t_v[g, pl.ds(k * L, L)]
                out_v[g, pl.ds(k * L, L)] = y0 * s0 + y1 * s1

    with jax.named_scope("writeback"):
        pltpu.sync_copy(out_v, out_hbm.at[:, pl.ds(base, qpw)])


@jax.jit
def _run(vt, ot, odomt):
    info = plsc.get_sparse_core_info()
    nc, ns = info.num_cores, info.num_subcores
    nw = nc * ns
    qpw = M // nw
    mesh = plsc.VectorSubcoreMesh(core_axis_name="c", subcore_axis_name="s")
    run = pl.kernel(
        functools.partial(_body, nc, qpw),
        out_type=jax.ShapeDtypeStruct((D, M), jnp.float32),
        mesh=mesh,
        compiler_params=pltpu.CompilerParams(
            needs_layout_passes=False, use_tc_tiling_on_sc=False),
        scratch_types=[pltpu.VMEM_SHARED((N,), jnp.float32)] * D + [
            pltpu.VMEM((N,), jnp.int32),
            pltpu.VMEM((qpw,), jnp.int32),
            pltpu.VMEM((qpw,), jnp.int32),
            pltpu.VMEM((qpw,), jnp.int32),
            pltpu.VMEM((qpw,), jnp.float32),
            pltpu.VMEM((qpw,), jnp.float32),
            pltpu.VMEM((D, qpw), jnp.float32),
            pltpu.VMEM((D, qpw), jnp.float32),
            pltpu.VMEM((D, CAP), jnp.float32),
            pltpu.VMEM((D, L), jnp.float32),
            pltpu.VMEM((D, qpw), jnp.float32),
            pltpu.SemaphoreType.DMA,
            pltpu.SemaphoreType.DMA,
            pltpu.SemaphoreType.DMA,
        ],
    )
    return run(vt, ot, odomt)


def kernel(valid_timestamps, odom_timestamps, odom):
    return _run(valid_timestamps, odom_timestamps, odom.T).T


# speedup vs baseline: 1.2726x; 1.0534x over previous
import functools

import jax
import jax.numpy as jnp
import numpy as np
from jax import lax
from jax.experimental import pallas as pl
from jax.experimental.pallas import tpu as pltpu
from jax.experimental.pallas import tpu_sc as plsc

M = 4096
N = 32768
L = 16
D = 12
CAP = 2048
SPEC = (M - 1) & ~7
IMAX = np.int32(2**31 - 1)


def _searchsorted(ot_v, target):
    pos = jnp.zeros((L,), jnp.int32)
    bit = N
    while bit >= 1:
        nxt = pos + bit
        ok = nxt <= N
        idx = jnp.minimum(nxt, N) - 1
        vals = plsc.load_gather(ot_v, [idx])
        pos = jnp.where(ok & (vals < target), nxt, pos)
        bit //= 2
    return pos


def _body(nc, qpw, vt_hbm, ot_hbm, odomt_hbm, out_hbm, *refs):
    cols_sh = refs[:D]
    (ot_v, vt_v, a_v, b_v, f_v, w0_v, w1_v, y0t_v, y1t_v, win_v, spec_v,
     out_v, sem0, sem1) = refs[D:]
    sid = lax.axis_index("s")
    wid = sid * nc + lax.axis_index("c")
    base = wid * qpw

    with jax.named_scope("stage_table"):
        pltpu.sync_copy(ot_hbm, ot_v)
        pltpu.sync_copy(vt_hbm.at[pl.ds(base, qpw)], vt_v)

    _scope = jax.named_scope("search")
    _scope.__enter__()
    rmin = jnp.full((L,), IMAX, jnp.int32)
    rmax = jnp.zeros((L,), jnp.int32)
    for k in range(qpw // L):
        vt16 = vt_v[pl.ds(k * L, L)]
        pL = _searchsorted(ot_v, vt16)
        wL = plsc.load_gather(ot_v, [jnp.maximum(pL, 1) - 1])
        wR = plsc.load_gather(ot_v, [jnp.minimum(pL, N - 1)])
        dL = vt16 - wL
        dR = jnp.where(pL < N, wR - vt16, IMAX)
        takeL = dL <= dR
        f = jnp.maximum(pL, 1) - 1
        for _ in range(2):
            fm1 = jnp.maximum(f, 1) - 1
            prev = plsc.load_gather(ot_v, [fm1])
            f = jnp.where((f > 0) & (prev == wL), fm1, f)
        fm1 = jnp.maximum(f, 1) - 1
        prev = plsc.load_gather(ot_v, [fm1])
        unresolved = (f > 0) & (prev == wL) & takeL
        f_v[pl.ds(0, L)] = f
        @pl.when(jnp.any(unresolved))
        def _():
            f_v[pl.ds(0, L)] = _searchsorted(ot_v, wL)
        first_wL = f_v[pl.ds(0, L)]
        ref = jnp.where(takeL, first_wL, pL)
        d = jnp.where(takeL, dL, -dR)
        step = (d > 0).astype(jnp.int32) - (d < 0).astype(jnp.int32)
        q = jnp.clip(ref + step, 0, M - 1)
        a = jnp.minimum(ref, q)
        b = jnp.maximum(ref, q)
        x0 = plsc.load_gather(ot_v, [a])
        x1 = plsc.load_gather(ot_v, [b])
        eq = x0 == x1
        x0f = x0.astype(jnp.float32)
        x1f = x1.astype(jnp.float32)
        vtf = vt16.astype(jnp.float32)
        denom = jnp.where(eq, jnp.float32(1.0), x1f - x0f)
        w0 = 1.0 - (vtf - x0f) / denom
        w1 = 1.0 - w0
        w0 = jnp.where(eq, jnp.float32(1.0), w0)
        w1 = jnp.where(eq, jnp.float32(0.0), w1)
        a_v[pl.ds(k * L, L)] = a
        b_v[pl.ds(k * L, L)] = b
        w0_v[pl.ds(k * L, L)] = w0
        w1_v[pl.ds(k * L, L)] = w1
        rmin = jnp.minimum(rmin, jnp.where(ref >= M, b, a))
        rmax = jnp.maximum(rmax, b)
    lo = jnp.min(rmin)
    hi = jnp.max(rmax)
    lo2 = pl.multiple_of(jnp.minimum(lo & ~7, N - CAP), 8)
    fits = hi < lo2 + CAP
    _scope.__exit__(None, None, None)

    with jax.named_scope("gather_rows"):
        @pl.when(fits)
        def _():
            pltpu.sync_copy(odomt_hbm.at[:, pl.ds(lo2, CAP)], win_v)
            pltpu.sync_copy(odomt_hbm.at[:, pl.ds(SPEC, L)], spec_v)
            for k in range(qpw // L):
                a16 = a_v[pl.ds(k * L, L)]
                b16 = b_v[pl.ds(k * L, L)] - lo2
                isclip = a16 == M - 1
                a16c = jnp.clip(a16 - lo2, 0, CAP - 1)
                for g in range(D):
                    g16 = jnp.full((L,), g, jnp.int32)
                    y0 = plsc.load_gather(win_v, [g16, a16c])
                    sp = spec_v[g][M - 1 - SPEC]
                    y0t_v[g, pl.ds(k * L, L)] = jnp.where(isclip, sp, y0)
                    y1t_v[g, pl.ds(k * L, L)] = plsc.load_gather(
                        win_v, [g16, b16])

        @pl.when(jnp.logical_not(fits))
        def _():
            for g in range(D):
                pltpu.sync_copy(odomt_hbm.at[g], cols_sh[g])
            handles = []
            for g in range(D):
                handles.append(
                    pltpu.async_copy(cols_sh[g].at[a_v], y0t_v.at[g], sem0))
                handles.append(
                    pltpu.async_copy(cols_sh[g].at[b_v], y1t_v.at[g], sem1))
            for h in handles:
                h.wait()

    with jax.named_scope("lerp"):
        for k in range(qpw // L):
            s0 = w0_v[pl.ds(k * L, L)]
            s1 = w1_v[pl.ds(k * L, L)]
            for g in range(D):
                y0 = y0t_v[g, pl.ds(k * L, L)]
                y1 = y1t_v[g, pl.ds(k * L, L)]
                out_v[g, pl.ds(k * L, L)] = y0 * s0 + y1 * s1

    with jax.named_scope("writeback"):
        pltpu.sync_copy(out_v, out_hbm.at[:, pl.ds(base, qpw)])


@jax.jit
def _run(vt, ot, odomt):
    info = plsc.get_sparse_core_info()
    nc, ns = info.num_cores, info.num_subcores
    nw = nc * ns
    qpw = M // nw
    mesh = plsc.VectorSubcoreMesh(core_axis_name="c", subcore_axis_name="s")
    run = pl.kernel(
        functools.partial(_body, nc, qpw),
        out_type=jax.ShapeDtypeStruct((D, M), jnp.float32),
        mesh=mesh,
        compiler_params=pltpu.CompilerParams(
            needs_layout_passes=False, use_tc_tiling_on_sc=False),
        scratch_types=[pltpu.VMEM_SHARED((N,), jnp.float32)] * D + [
            pltpu.VMEM((N,), jnp.int32),
            pltpu.VMEM((qpw,), jnp.int32),
            pltpu.VMEM((qpw,), jnp.int32),
            pltpu.VMEM((qpw,), jnp.int32),
            pltpu.VMEM((L,), jnp.int32),
            pltpu.VMEM((qpw,), jnp.float32),
            pltpu.VMEM((qpw,), jnp.float32),
            pltpu.VMEM((D, qpw), jnp.float32),
            pltpu.VMEM((D, qpw), jnp.float32),
            pltpu.VMEM((D, CAP), jnp.float32),
            pltpu.VMEM((D, L), jnp.float32),
            pltpu.VMEM((D, qpw), jnp.float32),
            pltpu.SemaphoreType.DMA,
            pltpu.SemaphoreType.DMA,
        ],
    )
    return run(vt, ot, odomt)


def kernel(valid_timestamps, odom_timestamps, odom):
    return _run(valid_timestamps, odom_timestamps, odom.T).T
